# Initial kernel scaffold; baseline (speedup 1.0000x reference)
import jax
import jax.numpy as jnp
from jax.experimental import pallas as pl


def kernel(x, y):
    raise NotImplementedError("write your pallas kernel here")


# speedup vs baseline: 1.8069x; 1.8069x over previous
import functools

import jax
import jax.numpy as jnp
from jax.experimental import pallas as pl
from jax.experimental.pallas import tpu as pltpu

SMOOTH = 0.1
PAD = 0
CONF = 1.0 - SMOOTH

BR = 256
BC = 2048


def _loss_kernel(x_ref, y_ref, out_ref, m_s, s_s, rs_s, xy_s, x0_s, *, c, ncb):
    i = pl.program_id(0)
    j = pl.program_id(1)

    @pl.when(j == 0)
    def _init():
        m_s[...] = jnp.full_like(m_s, -jnp.inf)
        s_s[...] = jnp.zeros_like(s_s)
        rs_s[...] = jnp.zeros_like(rs_s)
        xy_s[...] = jnp.zeros_like(xy_s)
        x0_s[...] = jnp.zeros_like(x0_s)

    xb = x_ref[...]
    col = j * BC + jax.lax.broadcasted_iota(jnp.int32, xb.shape, 1)
    inb = col < c
    xm = jnp.where(inb, xb, -jnp.inf)
    xz = jnp.where(inb, xb, 0.0)

    bm = jnp.max(xm, axis=1, keepdims=True)
    new_m = jnp.maximum(m_s[...], bm)
    corr = jnp.exp(m_s[...] - new_m)
    bs = jnp.sum(jnp.exp(xm - new_m), axis=1, keepdims=True)
    s_s[...] = s_s[...] * corr + bs
    m_s[...] = new_m

    rs_s[...] = rs_s[...] + jnp.sum(xz, axis=1, keepdims=True)

    yv = y_ref[...]
    xy_s[...] = xy_s[...] + jnp.sum(
        jnp.where(col == yv, xz, 0.0), axis=1, keepdims=True)
    x0_s[...] = x0_s[...] + jnp.sum(
        jnp.where(col == 0, xz, 0.0), axis=1, keepdims=True)

    @pl.when(j == ncb - 1)
    def _finish():
        eps = SMOOTH / (c - 2)
        k_const = SMOOTH * jnp.log(jnp.float32(eps)) + CONF * jnp.log(
            jnp.float32(CONF))
        lse = m_s[...] + jnp.log(s_s[...])
        logp0 = x0_s[...] - lse
        logpy = xy_s[...] - lse
        sum_logp = rs_s[...] - c * lse
        row = (k_const - eps * (sum_logp - logp0 - logpy) - CONF * logpy)
        row = jnp.where(y_ref[...] != PAD, row, 0.0)
        part = jnp.sum(row, keepdims=True)

        @pl.when(i == 0)
        def _():
            out_ref[...] = part

        @pl.when(i != 0)
        def _():
            out_ref[...] = out_ref[...] + part


@jax.jit
def kernel(x, y):
    b, c = x.shape
    ncb = pl.cdiv(c, BC)
    y2 = y.astype(jnp.int32).reshape(b, 1)
    out = pl.pallas_call(
        functools.partial(_loss_kernel, c=c, ncb=ncb),
        grid=(b // BR, ncb),
        in_specs=[
            pl.BlockSpec((BR, BC), lambda i, j: (i, j)),
            pl.BlockSpec((BR, 1), lambda i, j: (i, 0)),
        ],
        out_specs=pl.BlockSpec((1, 1), lambda i, j: (0, 0)),
        out_shape=jax.ShapeDtypeStruct((1, 1), jnp.float32),
        scratch_shapes=[pltpu.VMEM((BR, 1), jnp.float32) for _ in range(5)],
    )(x, y2)
    return out[0, 0]
